# baseline (speedup 1.0000x reference)
import jax
import jax.numpy as jnp
from jax import lax
from jax.experimental import pallas as pl
from jax.experimental.pallas import tpu as pltpu
from jax.experimental.pallas import tpu_sc as plsc

NUM_ENTITIES = 1000000
NUM_RELATIONS = 1000
EMBED_DIM = 64
PK = 128
BATCH = 16384

_info = plsc.get_sparse_core_info()
NC, NS, L = _info.num_cores, _info.num_subcores, _info.num_lanes
NW = NC * NS
RPW = BATCH // NW
CHUNK = 128
N_CHUNKS = RPW // CHUNK
D_VECS = EMBED_DIM // L


def _issue(entpk, relpk, eidx_v, ridx_v, ci, bufset, sem):
    e_v, r_v = bufset
    sl = pl.ds(ci * CHUNK, CHUNK)
    pltpu.async_copy(entpk.at[eidx_v.at[sl]], e_v, sem)
    pltpu.async_copy(relpk.at[ridx_v.at[sl]], r_v, sem)


def _drain(entpk, relpk, bufset, sem):
    e_v, r_v = bufset
    pltpu.make_async_copy(entpk.at[pl.ds(0, CHUNK)], e_v, sem).wait()
    pltpu.make_async_copy(relpk.at[pl.ds(0, CHUNK)], r_v, sem).wait()


def _compute(bufset, or_v, oi_v):
    e_v, r_v = bufset

    def row_body(row, carry):
        for cb in range(D_VECS):
            sl = pl.ds(cb * L, L)
            sli = pl.ds(EMBED_DIM + cb * L, L)
            a = e_v[row, sl]
            b = e_v[row, sli]
            cc = r_v[row, sl]
            d = r_v[row, sli]
            or_v[row, sl] = a * cc - b * d
            oi_v[row, sl] = a * d + b * cc
        return carry

    lax.fori_loop(0, CHUNK, row_body, 0)


def _body(e1_hbm, r_hbm, entpk, relpk, out_r, out_i,
          eidx_v, ridx_v, e0, r0, e1b, r1b, or_v, oi_v, sem0, sem1):
    wid = lax.axis_index("s") * NC + lax.axis_index("c")
    base = wid * RPW
    pltpu.sync_copy(e1_hbm.at[pl.ds(base, RPW)], eidx_v)
    pltpu.sync_copy(r_hbm.at[pl.ds(base, RPW)], ridx_v)

    bufs = ((e0, r0), (e1b, r1b))
    sems = (sem0, sem1)
    _issue(entpk, relpk, eidx_v, ridx_v, 0, bufs[0], sems[0])
    for ci in range(N_CHUNKS):
        par = ci % 2
        if ci + 1 < N_CHUNKS:
            _issue(entpk, relpk, eidx_v, ridx_v, ci + 1,
                   bufs[1 - par], sems[1 - par])
        _drain(entpk, relpk, bufs[par], sems[par])
        _compute(bufs[par], or_v, oi_v)
        off = base + ci * CHUNK
        pltpu.sync_copy(or_v, out_r.at[pl.ds(off, CHUNK)])
        pltpu.sync_copy(oi_v, out_i.at[pl.ds(off, CHUNK)])


@jax.jit
def kernel(e1, r, ent_real, ent_img, rel_real, rel_img):
    entpk = jnp.concatenate([ent_real, ent_img], axis=1)
    relpk = jnp.concatenate([rel_real, rel_img], axis=1)
    mesh = plsc.VectorSubcoreMesh(core_axis_name="c", subcore_axis_name="s")
    out_shape = jax.ShapeDtypeStruct((BATCH, EMBED_DIM), jnp.float32)
    buf = pltpu.VMEM((CHUNK, PK), jnp.float32)
    fn = pl.kernel(
        _body,
        out_type=(out_shape, out_shape),
        mesh=mesh,
        scratch_types=[
            pltpu.VMEM((RPW,), jnp.int32),
            pltpu.VMEM((RPW,), jnp.int32),
            buf, buf, buf, buf,
            pltpu.VMEM((CHUNK, EMBED_DIM), jnp.float32),
            pltpu.VMEM((CHUNK, EMBED_DIM), jnp.float32),
            pltpu.SemaphoreType.DMA,
            pltpu.SemaphoreType.DMA,
        ],
        compiler_params=pltpu.CompilerParams(
            use_tc_tiling_on_sc=True, needs_layout_passes=False),
    )
    return fn(e1, r, entpk, relpk)
